# baseline (speedup 1.0000x reference)
import functools

import jax
import jax.numpy as jnp
from jax import lax
from jax.experimental import pallas as pl
from jax.experimental.pallas import tpu as pltpu
from jax.experimental.pallas import tpu_sc as plsc

N_ITEMS = 1000000
N_RANKS = 20
BATCH = 16384
TOTAL = BATCH * N_RANKS

_info = plsc.get_sparse_core_info()
NC = _info.num_cores
NS = _info.num_subcores
NW = NC * NS
L = 16

COLS = BATCH // NW
GR = 4
NBUF = 3
UNROLL = 4

_mesh = plsc.VectorSubcoreMesh(core_axis_name="c", subcore_axis_name="s")


def _sigmoid(v):
    return 1.0 / (1.0 + jnp.exp(-v))


@functools.partial(
    pl.kernel,
    mesh=_mesh,
    out_type=jax.ShapeDtypeStruct((N_RANKS, BATCH), jnp.float32),
    scratch_types=[
        pltpu.VMEM((N_RANKS * COLS,), jnp.int32),
        pltpu.VMEM((N_RANKS * COLS,), jnp.float32),
        pltpu.VMEM((N_RANKS * COLS,), jnp.float32),
        pltpu.VMEM((48,), jnp.float32),
        pltpu.SemaphoreType.DMA,
        pltpu.SemaphoreType.DMA,
        pltpu.SemaphoreType.DMA,
        pltpu.SemaphoreType.DMA,
        pltpu.SemaphoreType.DMA,
        pltpu.SemaphoreType.DMA,
        pltpu.SemaphoreType.DMA,
        pltpu.SemaphoreType.DMA,
    ],
)
def _pbm_kernel(xt_hbm, exam_hbm, rel_hbm, out_hbm,
                idx_v, vals_v, resv, exam_v,
                gsem_a, gsem_b, gsem_c, gsem_d, gsem_e, gsem_f, isem, osem):
    wid = lax.axis_index("s") * NC + lax.axis_index("c")
    col0 = wid * COLS

    gsem = (gsem_a, gsem_b, gsem_c, gsem_d, gsem_e, gsem_f)
    rel_row = rel_hbm.at[0]

    idx_copies = [
        pltpu.async_copy(
            xt_hbm.at[r, pl.ds(col0, COLS)],
            idx_v.at[pl.ds(r * COLS, COLS)],
            isem,
        )
        for r in range(N_RANKS)
    ]

    def start_gather(q):
        for r in range(q * GR, (q + 1) * GR):
            idx_copies[r].wait()
        return pltpu.async_copy(
            rel_row.at[idx_v.at[pl.ds(q * GR * COLS, GR * COLS)]],
            vals_v.at[pl.ds(q * GR * COLS, GR * COLS)],
            gsem[q % NBUF],
        )

    gathers = [start_gather(q) for q in range(2)]
    pltpu.sync_copy(exam_hbm.at[0], exam_v.at[pl.ds(0, N_RANKS)])
    out_copies = []
    for r in range(N_RANKS):
        q = r // GR
        if r % GR == 0:
            if q + 2 < N_RANKS // GR:
                gathers.append(start_gather(q + 2))
            gathers[q].wait()
        ev = exam_v[pl.ds(r, L)]
        er = _sigmoid(jnp.full((L,), ev[0], jnp.float32))

        def group_body(g, _, r=r, er=er):
            for u in range(UNROLL):
                b = r * COLS + (g * UNROLL + u) * L
                resv[pl.ds(b, L)] = er * _sigmoid(vals_v[pl.ds(b, L)])
            return 0

        lax.fori_loop(0, COLS // (UNROLL * L), group_body, 0)
        out_copies.append(
            pltpu.async_copy(
                resv.at[pl.ds(r * COLS, COLS)],
                out_hbm.at[r, pl.ds(col0, COLS)],
                osem,
            )
        )

    for oc in out_copies:
        oc.wait()


def kernel(x, exam_table, rel_table):
    xt = x.T
    exam = exam_table.T
    rel = rel_table.T
    out_t = _pbm_kernel(xt, exam, rel)
    return out_t.T


# speedup vs baseline: 1.0276x; 1.0276x over previous
import functools

import jax
import jax.numpy as jnp
from jax import lax
from jax.experimental import pallas as pl
from jax.experimental.pallas import tpu as pltpu
from jax.experimental.pallas import tpu_sc as plsc

N_ITEMS = 1000000
N_RANKS = 20
BATCH = 16384
TOTAL = BATCH * N_RANKS

_info = plsc.get_sparse_core_info()
NC = _info.num_cores
NS = _info.num_subcores
NW = NC * NS
L = 16

COLS = BATCH // NW
NBUF = 6
UNROLL = 4

_mesh = plsc.VectorSubcoreMesh(core_axis_name="c", subcore_axis_name="s")


def _sigmoid(v):
    return 1.0 / (1.0 + jnp.exp(-v))


@functools.partial(
    pl.kernel,
    mesh=_mesh,
    out_type=jax.ShapeDtypeStruct((N_RANKS, BATCH), jnp.float32),
    scratch_types=[
        pltpu.VMEM((N_RANKS * COLS,), jnp.int32),
        pltpu.VMEM((N_RANKS * COLS,), jnp.float32),
        pltpu.VMEM((N_RANKS * COLS,), jnp.float32),
        pltpu.VMEM((48,), jnp.float32),
        pltpu.SemaphoreType.DMA,
        pltpu.SemaphoreType.DMA,
        pltpu.SemaphoreType.DMA,
        pltpu.SemaphoreType.DMA,
        pltpu.SemaphoreType.DMA,
        pltpu.SemaphoreType.DMA,
        pltpu.SemaphoreType.DMA,
        pltpu.SemaphoreType.DMA,
    ],
)
def _pbm_kernel(xt_hbm, exam_hbm, rel_hbm, out_hbm,
                idx_v, vals_v, resv, exam_v,
                gsem_a, gsem_b, gsem_c, gsem_d, gsem_e, gsem_f, isem, osem):
    wid = lax.axis_index("s") * NC + lax.axis_index("c")
    col0 = wid * COLS

    gsem = (gsem_a, gsem_b, gsem_c, gsem_d, gsem_e, gsem_f)
    rel_row = rel_hbm.at[0]

    idx_copies = [
        pltpu.async_copy(
            xt_hbm.at[r, pl.ds(col0, COLS)],
            idx_v.at[pl.ds(r * COLS, COLS)],
            isem,
        )
        for r in range(N_RANKS)
    ]

    def start_gather(r):
        idx_copies[r].wait()
        return pltpu.async_copy(
            rel_row.at[idx_v.at[pl.ds(r * COLS, COLS)]],
            vals_v.at[pl.ds(r * COLS, COLS)],
            gsem[r % NBUF],
        )

    gathers = [start_gather(r) for r in range(NBUF - 1)]
    pltpu.sync_copy(exam_hbm.at[0], exam_v.at[pl.ds(0, N_RANKS)])
    out_copies = []
    for r in range(N_RANKS):
        if r + NBUF - 1 < N_RANKS:
            gathers.append(start_gather(r + NBUF - 1))
        gathers[r].wait()
        ev = exam_v[pl.ds(r, L)]
        er = _sigmoid(jnp.full((L,), ev[0], jnp.float32))

        def group_body(g, _, r=r, er=er):
            for u in range(UNROLL):
                b = r * COLS + (g * UNROLL + u) * L
                resv[pl.ds(b, L)] = er * _sigmoid(vals_v[pl.ds(b, L)])
            return 0

        lax.fori_loop(0, COLS // (UNROLL * L), group_body, 0)
        out_copies.append(
            pltpu.async_copy(
                resv.at[pl.ds(r * COLS, COLS)],
                out_hbm.at[r, pl.ds(col0, COLS)],
                osem,
            )
        )

    for oc in out_copies:
        oc.wait()


def kernel(x, exam_table, rel_table):
    xt = x.T
    exam = exam_table.T
    rel = rel_table.T
    out_t = _pbm_kernel(xt, exam, rel)
    return out_t.T
